# Initial kernel scaffold; baseline (speedup 1.0000x reference)
import jax
import jax.numpy as jnp
from jax.experimental import pallas as pl


def kernel(sequence, group_id, table, conv_w, conv_b, proj_w, proj_b, score_w, score_b, ff_w, ff_b):
    raise NotImplementedError("write your pallas kernel here")


# speedup vs baseline: 26.7660x; 26.7660x over previous
import jax
import jax.numpy as jnp
from jax import lax
from jax.experimental import pallas as pl
from jax.experimental.pallas import tpu as pltpu

_B, _S, _D, _V, _NGRAM = 8, 512, 256, 384, 4
_BLOCKS = _NGRAM * (_NGRAM + 1) // 2
_NSEG = 64
_INTERPRET = False


def _body(seq_ref, gid_ref, table_ref, convw_ref, convb_ref, projw_ref,
          projb_ref, scorew_ref, ffw_ref, ffb_ref, out_ref,
          ep_ref, reps_ref):
    f32 = jnp.float32
    S, D, V = _S, _D, _V

    ident = (lax.broadcasted_iota(jnp.int32, (S, S), 0)
             == lax.broadcasted_iota(jnp.int32, (S, S), 1)).astype(f32)

    def to_col(v_row):
        return lax.dot_general(ident, v_row, (((1,), (1,)), ((), ())),
                               preferred_element_type=f32)

    seq_row = seq_ref[0].astype(f32)
    seq_col = to_col(seq_row)

    vlanes = lax.broadcasted_iota(jnp.int32, (S, V), 1).astype(f32)
    onehot = (seq_col == vlanes).astype(f32)
    emb = jnp.dot(onehot, table_ref[...], preferred_element_type=f32)

    ep_ref[0:S, :] = emb
    ep_ref[S:S + 8, :] = jnp.zeros((8, D), f32)
    cw = convw_ref[...]
    acc = ep_ref[0:S, :] * cw[0:1, :]
    for tap in range(1, _NGRAM):
        acc += ep_ref[tap:tap + S, :] * cw[tap:tap + 1, :]
    acc += convb_ref[...]
    embed = lax.dot_general(acc, projw_ref[...], (((1,), (1,)), ((), ())),
                            preferred_element_type=f32) + projb_ref[...]
    seq_is_pad = seq_col == 0.0
    embed = jnp.where(seq_is_pad, 0.0, embed)
    reps_ref[0] = embed

    def block_score(rep):
        return lax.dot_general(rep, scorew_ref[...], (((1,), (1,)), ((), ())),
                               preferred_element_type=f32)

    scores = [(block_score(embed), seq_is_pad)]

    glanes = lax.broadcasted_iota(jnp.int32, (S, _NSEG), 1).astype(f32)
    srows = lax.broadcasted_iota(jnp.int32, (S, _NSEG), 0).astype(f32)
    tri = (lax.broadcasted_iota(jnp.int32, (_NSEG, _NSEG), 0)
           < lax.broadcasted_iota(jnp.int32, (_NSEG, _NSEG), 1)).astype(f32)
    gid_all = gid_ref[0]
    for l in range(_BLOCKS - 1):
        g_row = gid_all[l:l + 1, :].astype(f32)
        g_col = to_col(g_row)
        gmax = jnp.max(g_row)
        is_pad = g_col == 0.0
        gadj = jnp.where(is_pad, gmax, g_col - 1.0)
        M = (gadj == glanes).astype(f32)
        counts = jnp.sum(M, axis=0, keepdims=True)
        Mn = M * (1.0 / jnp.maximum(counts, 1.0))
        e = jnp.where(is_pad, 0.0, embed)
        mean = lax.dot_general(Mn, e, (((0,), (0,)), ((), ())),
                               preferred_element_type=f32)
        cum = jnp.dot(counts, tri, preferred_element_type=f32)
        G = ((srows >= cum) & (srows < cum + counts)).astype(f32)
        rep = jnp.dot(G, mean, preferred_element_type=f32)
        reps_ref[l + 1] = rep
        scores.append((block_score(rep), is_pad))

    neg = -jnp.finfo(f32).max
    svals = [jnp.where(m, neg, s) for s, m in scores]
    mval = svals[0]
    for s in svals[1:]:
        mval = jnp.maximum(mval, s)
    exps = [jnp.exp(s - mval) for s in svals]
    den = exps[0]
    for e_ in exps[1:]:
        den = den + e_
    out = reps_ref[0] * (exps[0] / den)
    for l in range(1, _BLOCKS):
        out += reps_ref[l] * (exps[l] / den)

    y = lax.dot_general(out, ffw_ref[...], (((1,), (1,)), ((), ())),
                        preferred_element_type=f32) + ffb_ref[...]
    out_ref[0] = out + jnp.maximum(y, 0.0)


def kernel(sequence, group_id, table, conv_w, conv_b, proj_w, proj_b,
           score_w, score_b, ff_w, ff_b):
    B, S, D, V = _B, _S, _D, _V
    seq3 = sequence.reshape(B, 1, S)
    convw2 = conv_w[:, 0, :].T.reshape(_NGRAM, D)
    convb2 = conv_b.reshape(1, D)
    projb2 = proj_b.reshape(1, D)
    del score_b
    ffb2 = ff_b.reshape(1, D)

    grid = (B,)
    specs = [
        pl.BlockSpec((1, 1, S), lambda b: (b, 0, 0)),
        pl.BlockSpec((1, _BLOCKS - 1, S), lambda b: (b, 0, 0)),
        pl.BlockSpec((V, D), lambda b: (0, 0)),
        pl.BlockSpec((_NGRAM, D), lambda b: (0, 0)),
        pl.BlockSpec((1, D), lambda b: (0, 0)),
        pl.BlockSpec((D, D), lambda b: (0, 0)),
        pl.BlockSpec((1, D), lambda b: (0, 0)),
        pl.BlockSpec((1, D), lambda b: (0, 0)),
        pl.BlockSpec((D, D), lambda b: (0, 0)),
        pl.BlockSpec((1, D), lambda b: (0, 0)),
    ]
    out = pl.pallas_call(
        _body,
        grid=grid,
        in_specs=specs,
        out_specs=pl.BlockSpec((1, S, D), lambda b: (b, 0, 0)),
        out_shape=jax.ShapeDtypeStruct((B, S, D), jnp.float32),
        scratch_shapes=[
            pltpu.VMEM((S + 8, D), jnp.float32),
            pltpu.VMEM((_BLOCKS, S, D), jnp.float32),
        ],
        compiler_params=pltpu.CompilerParams(
            dimension_semantics=("arbitrary",)),
        interpret=_INTERPRET,
    )(seq3, group_id, table, convw2, convb2, proj_w, projb2,
      score_w, ff_w, ffb2)
    return out
